# baseline (speedup 1.0000x reference)
import jax
import jax.numpy as jnp
from jax import lax
from jax.experimental import pallas as pl
from jax.experimental.pallas import tpu as pltpu
from jax.experimental.pallas import tpu_sc as plsc

_N = 10000
_E = 320000
_DH = 64
_INV_TEMP = 1.0 / 0.6

_NSC = 2
_NTILE = 16
_CHUNK = 128
_NROWS = _E // _CHUNK
_NCHUNK = 157
_NPAD = _N + 112
_ZROWS = _NPAD // _NTILE


def _relu(x):
    return jnp.maximum(x, 0.0)


def _dot(a, b):
    return jnp.dot(a, b, preferred_element_type=jnp.float32)


def _sc_body(z_hbm, e0_hbm, e1_hbm, zero_hbm, out_hbm, sidx, didx,
             rows, msh, sem):
    c = lax.axis_index("c")
    s = lax.axis_index("s")
    row0 = s * 156 + jnp.minimum(s, 4)
    nchunk = jnp.where(s < 4, _NCHUNK, _NCHUNK - 1)

    def stage(e_hbm):
        pltpu.sync_copy(e_hbm.at[0, pl.ds(row0, _NCHUNK - 1)],
                        sidx.at[pl.ds(0, _NCHUNK - 1)])
        pltpu.sync_copy(e_hbm.at[1, pl.ds(row0, _NCHUNK - 1)],
                        didx.at[pl.ds(0, _NCHUNK - 1)])

        @pl.when(s < 4)
        def _():
            pltpu.sync_copy(e_hbm.at[0, pl.ds(row0 + _NCHUNK - 1, 1)],
                            sidx.at[pl.ds(_NCHUNK - 1, 1)])
            pltpu.sync_copy(e_hbm.at[1, pl.ds(row0 + _NCHUNK - 1, 1)],
                            didx.at[pl.ds(_NCHUNK - 1, 1)])

    @pl.when(c == 0)
    def _():
        stage(e0_hbm)

    @pl.when(c == 1)
    def _():
        stage(e1_hbm)

    pltpu.sync_copy(zero_hbm.at[pl.ds(s * _ZROWS, _ZROWS)],
                    msh.at[pl.ds(s * _ZROWS, _ZROWS)])
    plsc.subcore_barrier()

    def chunk(i, carry):
        pltpu.async_copy(z_hbm.at[sidx.at[i]], rows, sem).wait()
        pltpu.sync_copy(rows, msh.at[didx.at[i]], add=True)
        return carry

    lax.fori_loop(0, nchunk, chunk, 0)
    plsc.subcore_barrier()

    pltpu.sync_copy(msh.at[pl.ds(s * _ZROWS, _ZROWS)],
                    out_hbm.at[c, pl.ds(s * _ZROWS, _ZROWS)])


def _sc_segsum(z, e0, e1, zeros):
    fn = pl.kernel(
        _sc_body,
        out_type=jax.ShapeDtypeStruct((_NSC, _NPAD, _DH), jnp.float32),
        mesh=plsc.VectorSubcoreMesh(core_axis_name="c", subcore_axis_name="s",
                                    num_cores=_NSC, num_subcores=_NTILE),
        scratch_types=[
            pltpu.VMEM((_NCHUNK, _CHUNK), jnp.int32),
            pltpu.VMEM((_NCHUNK, _CHUNK), jnp.int32),
            pltpu.VMEM((_CHUNK, _DH), jnp.float32),
            pltpu.VMEM_SHARED((_NPAD, _DH), jnp.float32),
            pltpu.SemaphoreType.DMA,
        ],
        compiler_params=pltpu.CompilerParams(use_tc_tiling_on_sc=False),
    )
    return fn(z, e0, e1, zeros)


_BS = 2000
_GRID = _N // _BS


def _pre_body(x, w1, b1, w2, b2, w3, b3, wm, h_out, z_out):
    h = _relu(_dot(x[...], w1[...]) + b1[...])
    h = _relu(_dot(h, w2[...]) + b2[...])
    h = _relu(_dot(h, w3[...]) + b3[...])
    h_out[...] = h
    z_out[...] = _dot(h, wm[...])


def _gate_body(h, m3, ld0, ld1, v0, v1, wga, bga, wgb, bgb,
               a0_out, a1_out, y_out, s1_out, s2_out):
    m0 = jnp.clip(m3[0], -20.0, 20.0)
    m1 = jnp.clip(m3[1], -20.0, 20.0)
    hh = h[...]
    zc = jnp.zeros((hh.shape[0], 7), jnp.float32)
    gi0 = jnp.concatenate([hh, m0, ld0[...], zc], axis=1)
    gi1 = jnp.concatenate([hh, m1, ld1[...], zc], axis=1)
    g0 = _relu(_dot(gi0, wga[...]) + bga[...])
    g1 = _relu(_dot(gi1, wga[...]) + bga[...])
    s0 = (_dot(g0, wgb[...]) + bgb[...]) / 0.6
    s1 = (_dot(g1, wgb[...]) + bgb[...]) / 0.6
    mx = jnp.maximum(s0, s1)
    e0 = jnp.exp(s0 - mx)
    e1 = jnp.exp(s1 - mx)
    den = e0 + e1
    a0 = e0 / den * v0[...]
    a1 = e1 / den * v1[...]
    ssum = jnp.maximum(a0 + a1, 1e-12)
    a0 = a0 / ssum
    a1 = a1 / ssum
    a0 = jnp.maximum(a0, 1e-8)
    a1 = jnp.maximum(a1, 1e-8)
    ssum = jnp.maximum(a0 + a1, 1e-12)
    a0 = a0 / ssum
    a1 = a1 / ssum
    y = a0 * m0 + a1 * m1 + hh
    a0_out[...] = a0
    a1_out[...] = a1
    y_out[...] = y

    @pl.when(pl.program_id(0) == 0)
    def _():
        s1_out[...] = jnp.zeros_like(s1_out)
        s2_out[...] = jnp.zeros_like(s2_out)

    s1_out[...] += jnp.sum(y, axis=0, keepdims=True)
    s2_out[...] += jnp.sum(y * y).reshape(1, 1)


def _pairnorm(y_ref, s1_ref, s2_ref):
    mu = s1_ref[...] * (1.0 / _N)
    var = s2_ref[0, 0] * (1.0 / _N) - jnp.sum(mu * mu)
    msn = jnp.sqrt(var) + 1e-6
    return _relu((y_ref[...] - mu) / msn)


def _norm_mid_body(y, s1, s2, wmsg, h1_out, z1_out):
    h1 = _pairnorm(y, s1, s2)
    h1_out[...] = h1
    z1_out[...] = _dot(h1, wmsg[...])


def _norm_post_body(y, s1, s2, wh1, bh1, wh2, bh2, logit_out):
    h2 = _pairnorm(y, s1, s2)
    hh = _relu(_dot(h2, wh1[...]) + bh1[...])
    logit_out[...] = _dot(hh, wh2[...]) + bh2[...]


def _sds(shape):
    return jax.ShapeDtypeStruct(shape, jnp.float32)


def _row_spec(cols):
    return pl.BlockSpec((_BS, cols), lambda i: (i, 0))


def _full_spec(shape):
    nd = len(shape)
    return pl.BlockSpec(shape, lambda i: (0,) * nd)


def _gate_call(h, m3, ld0, ld1, v0, v1, wga, bga, wgb, bgb):
    return pl.pallas_call(
        _gate_body,
        grid=(_GRID,),
        in_specs=[
            _row_spec(_DH),
            pl.BlockSpec((_NSC, _BS, _DH), lambda i: (0, i, 0)),
            _row_spec(1), _row_spec(1), _row_spec(1), _row_spec(1),
            _full_spec((136, _DH)), _full_spec((1, _DH)),
            _full_spec((_DH, 1)), _full_spec((1, 1)),
        ],
        out_specs=[
            _row_spec(1), _row_spec(1), _row_spec(_DH),
            _full_spec((1, _DH)), _full_spec((1, 1)),
        ],
        out_shape=[_sds((_N, 1)), _sds((_N, 1)), _sds((_N, _DH)),
                   _sds((1, _DH)), _sds((1, 1))],
    )(h, m3, ld0, ld1, v0, v1, wga, bga, wgb, bgb)


def kernel(X, edge_index_0, edge_index_1, mask_0, mask_1, logdeg_0, logdeg_1,
           W_in1, b_in1, W_in2, b_in2, W_in3, b_in3, W_msg0, W_msg1,
           Wg0a, bg0a, Wg0b, bg0b, Wg1a, bg1a, Wg1b, bg1b, Wh1, bh1, Wh2, bh2):
    _io = jax.lax.iota(jnp.int32, _NROWS * _CHUNK)
    _seq = (_io % _N).reshape(1, _NROWS, _CHUNK)
    _rnd = ((_io * 7919 + 13) % _N).reshape(1, _NROWS, _CHUNK)
    e0 = jnp.concatenate([_seq, _rnd], axis=0)
    e1 = e0
    zeros = jnp.zeros((_NPAD, _DH), jnp.float32)

    ld0 = logdeg_0.reshape(_N, 1)
    ld1 = logdeg_1.reshape(_N, 1)
    v0 = mask_0.reshape(_N, 1)
    v1 = mask_1.reshape(_N, 1)
    b1 = b_in1.reshape(1, _DH)
    b2 = b_in2.reshape(1, _DH)
    b3 = b_in3.reshape(1, _DH)
    bg0 = bg0a.reshape(1, _DH)
    bg1 = bg1a.reshape(1, _DH)
    bg0b = bg0b.reshape(1, 1)
    bg1b = bg1b.reshape(1, 1)
    bh1r = bh1.reshape(1, _DH)
    bh2r = bh2.reshape(1, 1)
    wpad = jnp.zeros((7, _DH), jnp.float32)
    wg0p = jnp.concatenate([Wg0a, wpad], axis=0)
    wg1p = jnp.concatenate([Wg1a, wpad], axis=0)

    h, z0 = pl.pallas_call(
        _pre_body,
        grid=(_GRID,),
        in_specs=[
            pl.BlockSpec((_BS, 128), lambda i: (i, 0)),
            _full_spec((128, _DH)), _full_spec((1, _DH)),
            _full_spec((_DH, _DH)), _full_spec((1, _DH)),
            _full_spec((_DH, _DH)), _full_spec((1, _DH)),
            _full_spec((_DH, _DH)),
        ],
        out_specs=[_row_spec(_DH), _row_spec(_DH)],
        out_shape=[_sds((_N, _DH)), _sds((_N, _DH))],
    )(X, W_in1, b1, W_in2, b2, W_in3, b3, W_msg0)

    m0 = _sc_segsum(z0, e0, e1, zeros)

    a00, a01, y0, s1a, s2a = _gate_call(h, m0, ld0, ld1, v0, v1, wg0p, bg0,
                                        Wg0b, bg0b)
    h1, z1 = pl.pallas_call(
        _norm_mid_body,
        grid=(_GRID,),
        in_specs=[
            _row_spec(_DH), _full_spec((1, _DH)), _full_spec((1, 1)),
            _full_spec((_DH, _DH)),
        ],
        out_specs=[_row_spec(_DH), _row_spec(_DH)],
        out_shape=[_sds((_N, _DH)), _sds((_N, _DH))],
    )(y0, s1a, s2a, W_msg1)

    m1 = _sc_segsum(z1, e0, e1, zeros)

    a10, a11, y1, s1b, s2b = _gate_call(h1, m1, ld0, ld1, v0, v1, wg1p, bg1,
                                        Wg1b, bg1b)
    logits, = pl.pallas_call(
        _norm_post_body,
        grid=(_GRID,),
        in_specs=[
            _row_spec(_DH), _full_spec((1, _DH)), _full_spec((1, 1)),
            _full_spec((_DH, _DH)), _full_spec((1, _DH)),
            _full_spec((_DH, 1)), _full_spec((1, 1)),
        ],
        out_specs=[_row_spec(1)],
        out_shape=[_sds((_N, 1))],
    )(y1, s1b, s2b, Wh1, bh1r, Wh2, bh2r)

    alpha0 = jnp.concatenate([a00, a01], axis=1)
    alpha1 = jnp.concatenate([a10, a11], axis=1)
    return logits[:, 0], alpha0, alpha1


# speedup vs baseline: 1.2309x; 1.2309x over previous
import jax
import jax.numpy as jnp
from jax import lax
from jax.experimental import pallas as pl
from jax.experimental.pallas import tpu as pltpu
from jax.experimental.pallas import tpu_sc as plsc

_N = 10000
_E = 320000
_DH = 64
_INV_TEMP = 1.0 / 0.6

_NSC = 2
_NTILE = 16
_CHUNK = 256
_NROWS = _E // _CHUNK
_CBASE = _NROWS // _NTILE
_CEXT = _NROWS % _NTILE
_NCHUNK = _CBASE + 1
_NPAD = _N + 112
_ZROWS = _NPAD // _NTILE


def _relu(x):
    return jnp.maximum(x, 0.0)


def _dot(a, b):
    return jnp.dot(a, b, preferred_element_type=jnp.float32)


def _sc_body(z_hbm, e0_hbm, e1_hbm, zero_hbm, out_hbm, sidx, didx,
             rows, msh, sem):
    c = lax.axis_index("c")
    s = lax.axis_index("s")
    row0 = s * _CBASE + jnp.minimum(s, _CEXT)
    nchunk = jnp.where(s < _CEXT, _CBASE + 1, _CBASE)

    def stage(e_hbm):
        pltpu.sync_copy(e_hbm.at[0, pl.ds(row0, _CBASE)],
                        sidx.at[pl.ds(0, _CBASE)])
        pltpu.sync_copy(e_hbm.at[1, pl.ds(row0, _CBASE)],
                        didx.at[pl.ds(0, _CBASE)])

        @pl.when(s < _CEXT)
        def _():
            pltpu.sync_copy(e_hbm.at[0, pl.ds(row0 + _CBASE, 1)],
                            sidx.at[pl.ds(_CBASE, 1)])
            pltpu.sync_copy(e_hbm.at[1, pl.ds(row0 + _CBASE, 1)],
                            didx.at[pl.ds(_CBASE, 1)])

    @pl.when(c == 0)
    def _():
        stage(e0_hbm)

    @pl.when(c == 1)
    def _():
        stage(e1_hbm)

    pltpu.sync_copy(zero_hbm.at[pl.ds(s * _ZROWS, _ZROWS)],
                    msh.at[pl.ds(s * _ZROWS, _ZROWS)])
    plsc.subcore_barrier()

    def chunk(i, carry):
        pltpu.async_copy(z_hbm.at[sidx.at[i]], rows, sem).wait()
        pltpu.sync_copy(rows, msh.at[didx.at[i]], add=True)
        return carry

    lax.fori_loop(0, nchunk, chunk, 0)
    plsc.subcore_barrier()

    pltpu.sync_copy(msh.at[pl.ds(s * _ZROWS, _ZROWS)],
                    out_hbm.at[c, pl.ds(s * _ZROWS, _ZROWS)])


def _sc_segsum(z, e0, e1, zeros):
    fn = pl.kernel(
        _sc_body,
        out_type=jax.ShapeDtypeStruct((_NSC, _NPAD, _DH), jnp.float32),
        mesh=plsc.VectorSubcoreMesh(core_axis_name="c", subcore_axis_name="s",
                                    num_cores=_NSC, num_subcores=_NTILE),
        scratch_types=[
            pltpu.VMEM((_NCHUNK, _CHUNK), jnp.int32),
            pltpu.VMEM((_NCHUNK, _CHUNK), jnp.int32),
            pltpu.VMEM((_CHUNK, _DH), jnp.float32),
            pltpu.VMEM_SHARED((_NPAD, _DH), jnp.float32),
            pltpu.SemaphoreType.DMA,
        ],
        compiler_params=pltpu.CompilerParams(use_tc_tiling_on_sc=False),
    )
    return fn(z, e0, e1, zeros)


_BS = 2000
_GRID = _N // _BS


def _pre_body(x, w1, b1, w2, b2, w3, b3, wm, h_out, z_out):
    h = _relu(_dot(x[...], w1[...]) + b1[...])
    h = _relu(_dot(h, w2[...]) + b2[...])
    h = _relu(_dot(h, w3[...]) + b3[...])
    h_out[...] = h
    z_out[...] = _dot(h, wm[...])


def _gate_body(h, m3, ld0, ld1, v0, v1, wga, bga, wgb, bgb,
               a0_out, a1_out, y_out, s1_out, s2_out):
    m0 = jnp.clip(m3[0], -20.0, 20.0)
    m1 = jnp.clip(m3[1], -20.0, 20.0)
    hh = h[...]
    zc = jnp.zeros((hh.shape[0], 7), jnp.float32)
    gi0 = jnp.concatenate([hh, m0, ld0[...], zc], axis=1)
    gi1 = jnp.concatenate([hh, m1, ld1[...], zc], axis=1)
    g0 = _relu(_dot(gi0, wga[...]) + bga[...])
    g1 = _relu(_dot(gi1, wga[...]) + bga[...])
    s0 = (_dot(g0, wgb[...]) + bgb[...]) / 0.6
    s1 = (_dot(g1, wgb[...]) + bgb[...]) / 0.6
    mx = jnp.maximum(s0, s1)
    e0 = jnp.exp(s0 - mx)
    e1 = jnp.exp(s1 - mx)
    den = e0 + e1
    a0 = e0 / den * v0[...]
    a1 = e1 / den * v1[...]
    ssum = jnp.maximum(a0 + a1, 1e-12)
    a0 = a0 / ssum
    a1 = a1 / ssum
    a0 = jnp.maximum(a0, 1e-8)
    a1 = jnp.maximum(a1, 1e-8)
    ssum = jnp.maximum(a0 + a1, 1e-12)
    a0 = a0 / ssum
    a1 = a1 / ssum
    y = a0 * m0 + a1 * m1 + hh
    a0_out[...] = a0
    a1_out[...] = a1
    y_out[...] = y

    @pl.when(pl.program_id(0) == 0)
    def _():
        s1_out[...] = jnp.zeros_like(s1_out)
        s2_out[...] = jnp.zeros_like(s2_out)

    s1_out[...] += jnp.sum(y, axis=0, keepdims=True)
    s2_out[...] += jnp.sum(y * y).reshape(1, 1)


def _pairnorm(y_ref, s1_ref, s2_ref):
    mu = s1_ref[...] * (1.0 / _N)
    var = s2_ref[0, 0] * (1.0 / _N) - jnp.sum(mu * mu)
    msn = jnp.sqrt(var) + 1e-6
    return _relu((y_ref[...] - mu) / msn)


def _norm_mid_body(y, s1, s2, wmsg, h1_out, z1_out):
    h1 = _pairnorm(y, s1, s2)
    h1_out[...] = h1
    z1_out[...] = _dot(h1, wmsg[...])


def _norm_post_body(y, s1, s2, wh1, bh1, wh2, bh2, logit_out):
    h2 = _pairnorm(y, s1, s2)
    hh = _relu(_dot(h2, wh1[...]) + bh1[...])
    logit_out[...] = _dot(hh, wh2[...]) + bh2[...]


def _sds(shape):
    return jax.ShapeDtypeStruct(shape, jnp.float32)


def _row_spec(cols):
    return pl.BlockSpec((_BS, cols), lambda i: (i, 0))


def _full_spec(shape):
    nd = len(shape)
    return pl.BlockSpec(shape, lambda i: (0,) * nd)


def _gate_call(h, m3, ld0, ld1, v0, v1, wga, bga, wgb, bgb):
    return pl.pallas_call(
        _gate_body,
        grid=(_GRID,),
        in_specs=[
            _row_spec(_DH),
            pl.BlockSpec((_NSC, _BS, _DH), lambda i: (0, i, 0)),
            _row_spec(1), _row_spec(1), _row_spec(1), _row_spec(1),
            _full_spec((136, _DH)), _full_spec((1, _DH)),
            _full_spec((_DH, 1)), _full_spec((1, 1)),
        ],
        out_specs=[
            _row_spec(1), _row_spec(1), _row_spec(_DH),
            _full_spec((1, _DH)), _full_spec((1, 1)),
        ],
        out_shape=[_sds((_N, 1)), _sds((_N, 1)), _sds((_N, _DH)),
                   _sds((1, _DH)), _sds((1, 1))],
    )(h, m3, ld0, ld1, v0, v1, wga, bga, wgb, bgb)


def kernel(X, edge_index_0, edge_index_1, mask_0, mask_1, logdeg_0, logdeg_1,
           W_in1, b_in1, W_in2, b_in2, W_in3, b_in3, W_msg0, W_msg1,
           Wg0a, bg0a, Wg0b, bg0b, Wg1a, bg1a, Wg1b, bg1b, Wh1, bh1, Wh2, bh2):
    e0 = edge_index_0.astype(jnp.int32).reshape(2, _NROWS, _CHUNK)
    e1 = edge_index_1.astype(jnp.int32).reshape(2, _NROWS, _CHUNK)
    zeros = jnp.zeros((_NPAD, _DH), jnp.float32)

    ld0 = logdeg_0.reshape(_N, 1)
    ld1 = logdeg_1.reshape(_N, 1)
    v0 = mask_0.reshape(_N, 1)
    v1 = mask_1.reshape(_N, 1)
    b1 = b_in1.reshape(1, _DH)
    b2 = b_in2.reshape(1, _DH)
    b3 = b_in3.reshape(1, _DH)
    bg0 = bg0a.reshape(1, _DH)
    bg1 = bg1a.reshape(1, _DH)
    bg0b = bg0b.reshape(1, 1)
    bg1b = bg1b.reshape(1, 1)
    bh1r = bh1.reshape(1, _DH)
    bh2r = bh2.reshape(1, 1)
    wpad = jnp.zeros((7, _DH), jnp.float32)
    wg0p = jnp.concatenate([Wg0a, wpad], axis=0)
    wg1p = jnp.concatenate([Wg1a, wpad], axis=0)

    h, z0 = pl.pallas_call(
        _pre_body,
        grid=(_GRID,),
        in_specs=[
            pl.BlockSpec((_BS, 128), lambda i: (i, 0)),
            _full_spec((128, _DH)), _full_spec((1, _DH)),
            _full_spec((_DH, _DH)), _full_spec((1, _DH)),
            _full_spec((_DH, _DH)), _full_spec((1, _DH)),
            _full_spec((_DH, _DH)),
        ],
        out_specs=[_row_spec(_DH), _row_spec(_DH)],
        out_shape=[_sds((_N, _DH)), _sds((_N, _DH))],
    )(X, W_in1, b1, W_in2, b2, W_in3, b3, W_msg0)

    m0 = _sc_segsum(z0, e0, e1, zeros)

    a00, a01, y0, s1a, s2a = _gate_call(h, m0, ld0, ld1, v0, v1, wg0p, bg0,
                                        Wg0b, bg0b)
    h1, z1 = pl.pallas_call(
        _norm_mid_body,
        grid=(_GRID,),
        in_specs=[
            _row_spec(_DH), _full_spec((1, _DH)), _full_spec((1, 1)),
            _full_spec((_DH, _DH)),
        ],
        out_specs=[_row_spec(_DH), _row_spec(_DH)],
        out_shape=[_sds((_N, _DH)), _sds((_N, _DH))],
    )(y0, s1a, s2a, W_msg1)

    m1 = _sc_segsum(z1, e0, e1, zeros)

    a10, a11, y1, s1b, s2b = _gate_call(h1, m1, ld0, ld1, v0, v1, wg1p, bg1,
                                        Wg1b, bg1b)
    logits, = pl.pallas_call(
        _norm_post_body,
        grid=(_GRID,),
        in_specs=[
            _row_spec(_DH), _full_spec((1, _DH)), _full_spec((1, 1)),
            _full_spec((_DH, _DH)), _full_spec((1, _DH)),
            _full_spec((_DH, 1)), _full_spec((1, 1)),
        ],
        out_specs=[_row_spec(1)],
        out_shape=[_sds((_N, 1))],
    )(y1, s1b, s2b, Wh1, bh1r, Wh2, bh2r)

    alpha0 = jnp.concatenate([a00, a01], axis=1)
    alpha1 = jnp.concatenate([a10, a11], axis=1)
    return logits[:, 0], alpha0, alpha1


# speedup vs baseline: 1.3665x; 1.1101x over previous
import jax
import jax.numpy as jnp
from jax import lax
from jax.experimental import pallas as pl
from jax.experimental.pallas import tpu as pltpu
from jax.experimental.pallas import tpu_sc as plsc

_N = 10000
_E = 320000
_DH = 64
_INV_TEMP = 1.0 / 0.6

_NSC = 2
_NTILE = 16
_CHUNK = 512
_NROWS = _E // _CHUNK
_CBASE = _NROWS // _NTILE
_CEXT = _NROWS % _NTILE
_NCHUNK = _CBASE + 1
_NPAD = _N + 112
_ZROWS = _NPAD // _NTILE


def _relu(x):
    return jnp.maximum(x, 0.0)


def _dot(a, b):
    return jnp.dot(a, b, preferred_element_type=jnp.float32)


def _sc_body(z_hbm, e0_hbm, e1_hbm, zero_hbm, out_hbm, sidx, didx,
             rows, msh, sem):
    c = lax.axis_index("c")
    s = lax.axis_index("s")
    row0 = s * _CBASE + jnp.minimum(s, _CEXT)
    nchunk = jnp.where(s < _CEXT, _CBASE + 1, _CBASE)

    def stage(e_hbm):
        pltpu.sync_copy(e_hbm.at[0, pl.ds(row0, _CBASE)],
                        sidx.at[pl.ds(0, _CBASE)])
        pltpu.sync_copy(e_hbm.at[1, pl.ds(row0, _CBASE)],
                        didx.at[pl.ds(0, _CBASE)])

        @pl.when(s < _CEXT)
        def _():
            pltpu.sync_copy(e_hbm.at[0, pl.ds(row0 + _CBASE, 1)],
                            sidx.at[pl.ds(_CBASE, 1)])
            pltpu.sync_copy(e_hbm.at[1, pl.ds(row0 + _CBASE, 1)],
                            didx.at[pl.ds(_CBASE, 1)])

    @pl.when(c == 0)
    def _():
        stage(e0_hbm)

    @pl.when(c == 1)
    def _():
        stage(e1_hbm)

    pltpu.sync_copy(zero_hbm.at[pl.ds(s * _ZROWS, _ZROWS)],
                    msh.at[pl.ds(s * _ZROWS, _ZROWS)])
    plsc.subcore_barrier()

    def chunk(i, carry):
        pltpu.async_copy(z_hbm.at[sidx.at[i]], rows, sem).wait()
        pltpu.sync_copy(rows, msh.at[didx.at[i]], add=True)
        return carry

    lax.fori_loop(0, nchunk, chunk, 0)
    plsc.subcore_barrier()

    pltpu.sync_copy(msh.at[pl.ds(s * _ZROWS, _ZROWS)],
                    out_hbm.at[c, pl.ds(s * _ZROWS, _ZROWS)])


def _sc_segsum(z, e0, e1, zeros):
    fn = pl.kernel(
        _sc_body,
        out_type=jax.ShapeDtypeStruct((_NSC, _NPAD, _DH), jnp.float32),
        mesh=plsc.VectorSubcoreMesh(core_axis_name="c", subcore_axis_name="s",
                                    num_cores=_NSC, num_subcores=_NTILE),
        scratch_types=[
            pltpu.VMEM((_NCHUNK, _CHUNK), jnp.int32),
            pltpu.VMEM((_NCHUNK, _CHUNK), jnp.int32),
            pltpu.VMEM((_CHUNK, _DH), jnp.float32),
            pltpu.VMEM_SHARED((_NPAD, _DH), jnp.float32),
            pltpu.SemaphoreType.DMA,
        ],
        compiler_params=pltpu.CompilerParams(use_tc_tiling_on_sc=False),
    )
    return fn(z, e0, e1, zeros)


_BS = 2000
_GRID = _N // _BS


def _pre_body(x, w1, b1, w2, b2, w3, b3, wm, h_out, z_out):
    h = _relu(_dot(x[...], w1[...]) + b1[...])
    h = _relu(_dot(h, w2[...]) + b2[...])
    h = _relu(_dot(h, w3[...]) + b3[...])
    h_out[...] = h
    z_out[...] = _dot(h, wm[...])


def _gate_body(h, m3, ld0, ld1, v0, v1, wga, bga, wgb, bgb,
               a0_out, a1_out, y_out, s1_out, s2_out):
    m0 = jnp.clip(m3[0], -20.0, 20.0)
    m1 = jnp.clip(m3[1], -20.0, 20.0)
    hh = h[...]
    zc = jnp.zeros((hh.shape[0], 7), jnp.float32)
    gi0 = jnp.concatenate([hh, m0, ld0[...], zc], axis=1)
    gi1 = jnp.concatenate([hh, m1, ld1[...], zc], axis=1)
    g0 = _relu(_dot(gi0, wga[...]) + bga[...])
    g1 = _relu(_dot(gi1, wga[...]) + bga[...])
    s0 = (_dot(g0, wgb[...]) + bgb[...]) / 0.6
    s1 = (_dot(g1, wgb[...]) + bgb[...]) / 0.6
    mx = jnp.maximum(s0, s1)
    e0 = jnp.exp(s0 - mx)
    e1 = jnp.exp(s1 - mx)
    den = e0 + e1
    a0 = e0 / den * v0[...]
    a1 = e1 / den * v1[...]
    ssum = jnp.maximum(a0 + a1, 1e-12)
    a0 = a0 / ssum
    a1 = a1 / ssum
    a0 = jnp.maximum(a0, 1e-8)
    a1 = jnp.maximum(a1, 1e-8)
    ssum = jnp.maximum(a0 + a1, 1e-12)
    a0 = a0 / ssum
    a1 = a1 / ssum
    y = a0 * m0 + a1 * m1 + hh
    a0_out[...] = a0
    a1_out[...] = a1
    y_out[...] = y

    @pl.when(pl.program_id(0) == 0)
    def _():
        s1_out[...] = jnp.zeros_like(s1_out)
        s2_out[...] = jnp.zeros_like(s2_out)

    s1_out[...] += jnp.sum(y, axis=0, keepdims=True)
    s2_out[...] += jnp.sum(y * y).reshape(1, 1)


def _pairnorm(y_ref, s1_ref, s2_ref):
    mu = s1_ref[...] * (1.0 / _N)
    var = s2_ref[0, 0] * (1.0 / _N) - jnp.sum(mu * mu)
    msn = jnp.sqrt(var) + 1e-6
    return _relu((y_ref[...] - mu) / msn)


def _norm_mid_body(y, s1, s2, wmsg, h1_out, z1_out):
    h1 = _pairnorm(y, s1, s2)
    h1_out[...] = h1
    z1_out[...] = _dot(h1, wmsg[...])


def _norm_post_body(y, s1, s2, wh1, bh1, wh2, bh2, logit_out):
    h2 = _pairnorm(y, s1, s2)
    hh = _relu(_dot(h2, wh1[...]) + bh1[...])
    logit_out[...] = _dot(hh, wh2[...]) + bh2[...]


def _sds(shape):
    return jax.ShapeDtypeStruct(shape, jnp.float32)


def _row_spec(cols):
    return pl.BlockSpec((_BS, cols), lambda i: (i, 0))


def _full_spec(shape):
    nd = len(shape)
    return pl.BlockSpec(shape, lambda i: (0,) * nd)


def _gate_call(h, m3, ld0, ld1, v0, v1, wga, bga, wgb, bgb):
    return pl.pallas_call(
        _gate_body,
        grid=(_GRID,),
        in_specs=[
            _row_spec(_DH),
            pl.BlockSpec((_NSC, _BS, _DH), lambda i: (0, i, 0)),
            _row_spec(1), _row_spec(1), _row_spec(1), _row_spec(1),
            _full_spec((136, _DH)), _full_spec((1, _DH)),
            _full_spec((_DH, 1)), _full_spec((1, 1)),
        ],
        out_specs=[
            _row_spec(1), _row_spec(1), _row_spec(_DH),
            _full_spec((1, _DH)), _full_spec((1, 1)),
        ],
        out_shape=[_sds((_N, 1)), _sds((_N, 1)), _sds((_N, _DH)),
                   _sds((1, _DH)), _sds((1, 1))],
    )(h, m3, ld0, ld1, v0, v1, wga, bga, wgb, bgb)


def kernel(X, edge_index_0, edge_index_1, mask_0, mask_1, logdeg_0, logdeg_1,
           W_in1, b_in1, W_in2, b_in2, W_in3, b_in3, W_msg0, W_msg1,
           Wg0a, bg0a, Wg0b, bg0b, Wg1a, bg1a, Wg1b, bg1b, Wh1, bh1, Wh2, bh2):
    e0 = edge_index_0.astype(jnp.int32).reshape(2, _NROWS, _CHUNK)
    e1 = edge_index_1.astype(jnp.int32).reshape(2, _NROWS, _CHUNK)
    zeros = jnp.zeros((_NPAD, _DH), jnp.float32)

    ld0 = logdeg_0.reshape(_N, 1)
    ld1 = logdeg_1.reshape(_N, 1)
    v0 = mask_0.reshape(_N, 1)
    v1 = mask_1.reshape(_N, 1)
    b1 = b_in1.reshape(1, _DH)
    b2 = b_in2.reshape(1, _DH)
    b3 = b_in3.reshape(1, _DH)
    bg0 = bg0a.reshape(1, _DH)
    bg1 = bg1a.reshape(1, _DH)
    bg0b = bg0b.reshape(1, 1)
    bg1b = bg1b.reshape(1, 1)
    bh1r = bh1.reshape(1, _DH)
    bh2r = bh2.reshape(1, 1)
    wpad = jnp.zeros((7, _DH), jnp.float32)
    wg0p = jnp.concatenate([Wg0a, wpad], axis=0)
    wg1p = jnp.concatenate([Wg1a, wpad], axis=0)

    h, z0 = pl.pallas_call(
        _pre_body,
        grid=(_GRID,),
        in_specs=[
            pl.BlockSpec((_BS, 128), lambda i: (i, 0)),
            _full_spec((128, _DH)), _full_spec((1, _DH)),
            _full_spec((_DH, _DH)), _full_spec((1, _DH)),
            _full_spec((_DH, _DH)), _full_spec((1, _DH)),
            _full_spec((_DH, _DH)),
        ],
        out_specs=[_row_spec(_DH), _row_spec(_DH)],
        out_shape=[_sds((_N, _DH)), _sds((_N, _DH))],
    )(X, W_in1, b1, W_in2, b2, W_in3, b3, W_msg0)

    m0 = _sc_segsum(z0, e0, e1, zeros)

    a00, a01, y0, s1a, s2a = _gate_call(h, m0, ld0, ld1, v0, v1, wg0p, bg0,
                                        Wg0b, bg0b)
    h1, z1 = pl.pallas_call(
        _norm_mid_body,
        grid=(_GRID,),
        in_specs=[
            _row_spec(_DH), _full_spec((1, _DH)), _full_spec((1, 1)),
            _full_spec((_DH, _DH)),
        ],
        out_specs=[_row_spec(_DH), _row_spec(_DH)],
        out_shape=[_sds((_N, _DH)), _sds((_N, _DH))],
    )(y0, s1a, s2a, W_msg1)

    m1 = _sc_segsum(z1, e0, e1, zeros)

    a10, a11, y1, s1b, s2b = _gate_call(h1, m1, ld0, ld1, v0, v1, wg1p, bg1,
                                        Wg1b, bg1b)
    logits, = pl.pallas_call(
        _norm_post_body,
        grid=(_GRID,),
        in_specs=[
            _row_spec(_DH), _full_spec((1, _DH)), _full_spec((1, 1)),
            _full_spec((_DH, _DH)), _full_spec((1, _DH)),
            _full_spec((_DH, 1)), _full_spec((1, 1)),
        ],
        out_specs=[_row_spec(1)],
        out_shape=[_sds((_N, 1))],
    )(y1, s1b, s2b, Wh1, bh1r, Wh2, bh2r)

    alpha0 = jnp.concatenate([a00, a01], axis=1)
    alpha1 = jnp.concatenate([a10, a11], axis=1)
    return logits[:, 0], alpha0, alpha1


# speedup vs baseline: 1.3858x; 1.0141x over previous
import jax
import jax.numpy as jnp
from jax import lax
from jax.experimental import pallas as pl
from jax.experimental.pallas import tpu as pltpu
from jax.experimental.pallas import tpu_sc as plsc

_N = 10000
_E = 320000
_DH = 64
_INV_TEMP = 1.0 / 0.6

_NSC = 2
_NTILE = 16
_CHUNK = 640
_NROWS = _E // _CHUNK
_CBASE = _NROWS // _NTILE
_CEXT = _NROWS % _NTILE
_NCHUNK = _CBASE + 1
_NPAD = _N + 112
_ZROWS = _NPAD // _NTILE


def _relu(x):
    return jnp.maximum(x, 0.0)


def _dot(a, b):
    return jnp.dot(a, b, preferred_element_type=jnp.float32)


def _sc_body(z_hbm, e0_hbm, e1_hbm, zero_hbm, out_hbm, sidx, didx,
             rows, msh, sem):
    c = lax.axis_index("c")
    s = lax.axis_index("s")
    row0 = s * _CBASE + jnp.minimum(s, _CEXT)
    nchunk = jnp.where(s < _CEXT, _CBASE + 1, _CBASE)

    def stage(e_hbm):
        pltpu.sync_copy(e_hbm.at[0, pl.ds(row0, _CBASE)],
                        sidx.at[pl.ds(0, _CBASE)])
        pltpu.sync_copy(e_hbm.at[1, pl.ds(row0, _CBASE)],
                        didx.at[pl.ds(0, _CBASE)])

        @pl.when(s < _CEXT)
        def _():
            pltpu.sync_copy(e_hbm.at[0, pl.ds(row0 + _CBASE, 1)],
                            sidx.at[pl.ds(_CBASE, 1)])
            pltpu.sync_copy(e_hbm.at[1, pl.ds(row0 + _CBASE, 1)],
                            didx.at[pl.ds(_CBASE, 1)])

    @pl.when(c == 0)
    def _():
        stage(e0_hbm)

    @pl.when(c == 1)
    def _():
        stage(e1_hbm)

    pltpu.sync_copy(zero_hbm.at[pl.ds(s * _ZROWS, _ZROWS)],
                    msh.at[pl.ds(s * _ZROWS, _ZROWS)])
    plsc.subcore_barrier()

    def chunk(i, carry):
        pltpu.async_copy(z_hbm.at[sidx.at[i]], rows, sem).wait()
        pltpu.sync_copy(rows, msh.at[didx.at[i]], add=True)
        return carry

    lax.fori_loop(0, nchunk, chunk, 0)
    plsc.subcore_barrier()

    pltpu.sync_copy(msh.at[pl.ds(s * _ZROWS, _ZROWS)],
                    out_hbm.at[c, pl.ds(s * _ZROWS, _ZROWS)])


def _sc_segsum(z, e0, e1, zeros):
    fn = pl.kernel(
        _sc_body,
        out_type=jax.ShapeDtypeStruct((_NSC, _NPAD, _DH), jnp.float32),
        mesh=plsc.VectorSubcoreMesh(core_axis_name="c", subcore_axis_name="s",
                                    num_cores=_NSC, num_subcores=_NTILE),
        scratch_types=[
            pltpu.VMEM((_NCHUNK, _CHUNK), jnp.int32),
            pltpu.VMEM((_NCHUNK, _CHUNK), jnp.int32),
            pltpu.VMEM((_CHUNK, _DH), jnp.float32),
            pltpu.VMEM_SHARED((_NPAD, _DH), jnp.float32),
            pltpu.SemaphoreType.DMA,
        ],
        compiler_params=pltpu.CompilerParams(use_tc_tiling_on_sc=False),
    )
    return fn(z, e0, e1, zeros)


_BS = 2000
_GRID = _N // _BS


def _pre_body(x, w1, b1, w2, b2, w3, b3, wm, h_out, z_out):
    h = _relu(_dot(x[...], w1[...]) + b1[...])
    h = _relu(_dot(h, w2[...]) + b2[...])
    h = _relu(_dot(h, w3[...]) + b3[...])
    h_out[...] = h
    z_out[...] = _dot(h, wm[...])


def _gate_body(h, m3, ld0, ld1, v0, v1, wga, bga, wgb, bgb,
               a0_out, a1_out, y_out, s1_out, s2_out):
    m0 = jnp.clip(m3[0], -20.0, 20.0)
    m1 = jnp.clip(m3[1], -20.0, 20.0)
    hh = h[...]
    zc = jnp.zeros((hh.shape[0], 7), jnp.float32)
    gi0 = jnp.concatenate([hh, m0, ld0[...], zc], axis=1)
    gi1 = jnp.concatenate([hh, m1, ld1[...], zc], axis=1)
    g0 = _relu(_dot(gi0, wga[...]) + bga[...])
    g1 = _relu(_dot(gi1, wga[...]) + bga[...])
    s0 = (_dot(g0, wgb[...]) + bgb[...]) / 0.6
    s1 = (_dot(g1, wgb[...]) + bgb[...]) / 0.6
    mx = jnp.maximum(s0, s1)
    e0 = jnp.exp(s0 - mx)
    e1 = jnp.exp(s1 - mx)
    den = e0 + e1
    a0 = e0 / den * v0[...]
    a1 = e1 / den * v1[...]
    ssum = jnp.maximum(a0 + a1, 1e-12)
    a0 = a0 / ssum
    a1 = a1 / ssum
    a0 = jnp.maximum(a0, 1e-8)
    a1 = jnp.maximum(a1, 1e-8)
    ssum = jnp.maximum(a0 + a1, 1e-12)
    a0 = a0 / ssum
    a1 = a1 / ssum
    y = a0 * m0 + a1 * m1 + hh
    a0_out[...] = a0
    a1_out[...] = a1
    y_out[...] = y

    @pl.when(pl.program_id(0) == 0)
    def _():
        s1_out[...] = jnp.zeros_like(s1_out)
        s2_out[...] = jnp.zeros_like(s2_out)

    s1_out[...] += jnp.sum(y, axis=0, keepdims=True)
    s2_out[...] += jnp.sum(y * y).reshape(1, 1)


def _pairnorm(y_ref, s1_ref, s2_ref):
    mu = s1_ref[...] * (1.0 / _N)
    var = s2_ref[0, 0] * (1.0 / _N) - jnp.sum(mu * mu)
    msn = jnp.sqrt(var) + 1e-6
    return _relu((y_ref[...] - mu) / msn)


def _norm_mid_body(y, s1, s2, wmsg, h1_out, z1_out):
    h1 = _pairnorm(y, s1, s2)
    h1_out[...] = h1
    z1_out[...] = _dot(h1, wmsg[...])


def _norm_post_body(y, s1, s2, wh1, bh1, wh2, bh2, logit_out):
    h2 = _pairnorm(y, s1, s2)
    hh = _relu(_dot(h2, wh1[...]) + bh1[...])
    logit_out[...] = _dot(hh, wh2[...]) + bh2[...]


def _sds(shape):
    return jax.ShapeDtypeStruct(shape, jnp.float32)


def _row_spec(cols):
    return pl.BlockSpec((_BS, cols), lambda i: (i, 0))


def _full_spec(shape):
    nd = len(shape)
    return pl.BlockSpec(shape, lambda i: (0,) * nd)


def _gate_call(h, m3, ld0, ld1, v0, v1, wga, bga, wgb, bgb):
    return pl.pallas_call(
        _gate_body,
        grid=(_GRID,),
        in_specs=[
            _row_spec(_DH),
            pl.BlockSpec((_NSC, _BS, _DH), lambda i: (0, i, 0)),
            _row_spec(1), _row_spec(1), _row_spec(1), _row_spec(1),
            _full_spec((136, _DH)), _full_spec((1, _DH)),
            _full_spec((_DH, 1)), _full_spec((1, 1)),
        ],
        out_specs=[
            _row_spec(1), _row_spec(1), _row_spec(_DH),
            _full_spec((1, _DH)), _full_spec((1, 1)),
        ],
        out_shape=[_sds((_N, 1)), _sds((_N, 1)), _sds((_N, _DH)),
                   _sds((1, _DH)), _sds((1, 1))],
    )(h, m3, ld0, ld1, v0, v1, wga, bga, wgb, bgb)


def kernel(X, edge_index_0, edge_index_1, mask_0, mask_1, logdeg_0, logdeg_1,
           W_in1, b_in1, W_in2, b_in2, W_in3, b_in3, W_msg0, W_msg1,
           Wg0a, bg0a, Wg0b, bg0b, Wg1a, bg1a, Wg1b, bg1b, Wh1, bh1, Wh2, bh2):
    e0 = edge_index_0.astype(jnp.int32).reshape(2, _NROWS, _CHUNK)
    e1 = edge_index_1.astype(jnp.int32).reshape(2, _NROWS, _CHUNK)
    zeros = jnp.zeros((_NPAD, _DH), jnp.float32)

    ld0 = logdeg_0.reshape(_N, 1)
    ld1 = logdeg_1.reshape(_N, 1)
    v0 = mask_0.reshape(_N, 1)
    v1 = mask_1.reshape(_N, 1)
    b1 = b_in1.reshape(1, _DH)
    b2 = b_in2.reshape(1, _DH)
    b3 = b_in3.reshape(1, _DH)
    bg0 = bg0a.reshape(1, _DH)
    bg1 = bg1a.reshape(1, _DH)
    bg0b = bg0b.reshape(1, 1)
    bg1b = bg1b.reshape(1, 1)
    bh1r = bh1.reshape(1, _DH)
    bh2r = bh2.reshape(1, 1)
    wpad = jnp.zeros((7, _DH), jnp.float32)
    wg0p = jnp.concatenate([Wg0a, wpad], axis=0)
    wg1p = jnp.concatenate([Wg1a, wpad], axis=0)

    h, z0 = pl.pallas_call(
        _pre_body,
        grid=(_GRID,),
        in_specs=[
            pl.BlockSpec((_BS, 128), lambda i: (i, 0)),
            _full_spec((128, _DH)), _full_spec((1, _DH)),
            _full_spec((_DH, _DH)), _full_spec((1, _DH)),
            _full_spec((_DH, _DH)), _full_spec((1, _DH)),
            _full_spec((_DH, _DH)),
        ],
        out_specs=[_row_spec(_DH), _row_spec(_DH)],
        out_shape=[_sds((_N, _DH)), _sds((_N, _DH))],
    )(X, W_in1, b1, W_in2, b2, W_in3, b3, W_msg0)

    m0 = _sc_segsum(z0, e0, e1, zeros)

    a00, a01, y0, s1a, s2a = _gate_call(h, m0, ld0, ld1, v0, v1, wg0p, bg0,
                                        Wg0b, bg0b)
    h1, z1 = pl.pallas_call(
        _norm_mid_body,
        grid=(_GRID,),
        in_specs=[
            _row_spec(_DH), _full_spec((1, _DH)), _full_spec((1, 1)),
            _full_spec((_DH, _DH)),
        ],
        out_specs=[_row_spec(_DH), _row_spec(_DH)],
        out_shape=[_sds((_N, _DH)), _sds((_N, _DH))],
    )(y0, s1a, s2a, W_msg1)

    m1 = _sc_segsum(z1, e0, e1, zeros)

    a10, a11, y1, s1b, s2b = _gate_call(h1, m1, ld0, ld1, v0, v1, wg1p, bg1,
                                        Wg1b, bg1b)
    logits, = pl.pallas_call(
        _norm_post_body,
        grid=(_GRID,),
        in_specs=[
            _row_spec(_DH), _full_spec((1, _DH)), _full_spec((1, 1)),
            _full_spec((_DH, _DH)), _full_spec((1, _DH)),
            _full_spec((_DH, 1)), _full_spec((1, 1)),
        ],
        out_specs=[_row_spec(1)],
        out_shape=[_sds((_N, 1))],
    )(y1, s1b, s2b, Wh1, bh1r, Wh2, bh2r)

    alpha0 = jnp.concatenate([a00, a01], axis=1)
    alpha1 = jnp.concatenate([a10, a11], axis=1)
    return logits[:, 0], alpha0, alpha1


# speedup vs baseline: 1.5197x; 1.0967x over previous
import jax
import jax.numpy as jnp
from jax import lax
from jax.experimental import pallas as pl
from jax.experimental.pallas import tpu as pltpu
from jax.experimental.pallas import tpu_sc as plsc

_N = 10000
_E = 320000
_DH = 64
_INV_TEMP = 1.0 / 0.6

_NSC = 2
_NTILE = 16
_CHUNK = 256
_NROWS = _E // _CHUNK
_CBASE = _NROWS // _NTILE
_CEXT = _NROWS % _NTILE
_NCHUNK = _CBASE + 1
_NPAD = _N + 112
_ZROWS = _NPAD // _NTILE


def _relu(x):
    return jnp.maximum(x, 0.0)


def _dot(a, b):
    return jnp.dot(a, b, preferred_element_type=jnp.float32)


def _sc_body(z_hbm, e0_hbm, e1_hbm, zero_hbm, out_hbm, sidx, didx,
             rows, rows_b, msh, sem, sem_b):
    c = lax.axis_index("c")
    s = lax.axis_index("s")
    row0 = s * _CBASE + jnp.minimum(s, _CEXT)
    nchunk = jnp.where(s < _CEXT, _CBASE + 1, _CBASE)

    def stage(e_hbm):
        pltpu.sync_copy(e_hbm.at[0, pl.ds(row0, _CBASE)],
                        sidx.at[pl.ds(0, _CBASE)])
        pltpu.sync_copy(e_hbm.at[1, pl.ds(row0, _CBASE)],
                        didx.at[pl.ds(0, _CBASE)])

        @pl.when(s < _CEXT)
        def _():
            pltpu.sync_copy(e_hbm.at[0, pl.ds(row0 + _CBASE, 1)],
                            sidx.at[pl.ds(_CBASE, 1)])
            pltpu.sync_copy(e_hbm.at[1, pl.ds(row0 + _CBASE, 1)],
                            didx.at[pl.ds(_CBASE, 1)])

    @pl.when(c == 0)
    def _():
        stage(e0_hbm)

    @pl.when(c == 1)
    def _():
        stage(e1_hbm)

    pltpu.sync_copy(zero_hbm.at[pl.ds(s * _ZROWS, _ZROWS)],
                    msh.at[pl.ds(s * _ZROWS, _ZROWS)])
    plsc.subcore_barrier()

    pltpu.async_copy(z_hbm.at[sidx.at[0]], rows, sem)

    def pair(p, carry):
        i0 = 2 * p
        pltpu.make_async_copy(z_hbm.at[sidx.at[i0]], rows, sem).wait()
        pltpu.async_copy(z_hbm.at[sidx.at[i0 + 1]], rows_b, sem_b)
        pltpu.sync_copy(rows, msh.at[didx.at[i0]], add=True)
        pltpu.make_async_copy(z_hbm.at[sidx.at[i0 + 1]], rows_b, sem_b).wait()
        pltpu.async_copy(z_hbm.at[sidx.at[lax.rem(i0 + 2, _CBASE)]], rows, sem)
        pltpu.sync_copy(rows_b, msh.at[didx.at[i0 + 1]], add=True)
        return carry

    lax.fori_loop(0, _CBASE // 2, pair, 0)
    pltpu.make_async_copy(z_hbm.at[sidx.at[0]], rows, sem).wait()

    @pl.when(s < _CEXT)
    def _():
        pltpu.async_copy(z_hbm.at[sidx.at[_CBASE]], rows, sem).wait()
        pltpu.sync_copy(rows, msh.at[didx.at[_CBASE]], add=True)

    plsc.subcore_barrier()

    pltpu.sync_copy(msh.at[pl.ds(s * _ZROWS, _ZROWS)],
                    out_hbm.at[c, pl.ds(s * _ZROWS, _ZROWS)])


def _sc_segsum(z, e0, e1, zeros):
    fn = pl.kernel(
        _sc_body,
        out_type=jax.ShapeDtypeStruct((_NSC, _NPAD, _DH), jnp.float32),
        mesh=plsc.VectorSubcoreMesh(core_axis_name="c", subcore_axis_name="s",
                                    num_cores=_NSC, num_subcores=_NTILE),
        scratch_types=[
            pltpu.VMEM((_NCHUNK, _CHUNK), jnp.int32),
            pltpu.VMEM((_NCHUNK, _CHUNK), jnp.int32),
            pltpu.VMEM((_CHUNK, _DH), jnp.float32),
            pltpu.VMEM((_CHUNK, _DH), jnp.float32),
            pltpu.VMEM_SHARED((_NPAD, _DH), jnp.float32),
            pltpu.SemaphoreType.DMA,
            pltpu.SemaphoreType.DMA,
        ],
        compiler_params=pltpu.CompilerParams(use_tc_tiling_on_sc=False),
    )
    return fn(z, e0, e1, zeros)


_BS = 2000
_GRID = _N // _BS


def _pre_body(x, w1, b1, w2, b2, w3, b3, wm, h_out, z_out):
    h = _relu(_dot(x[...], w1[...]) + b1[...])
    h = _relu(_dot(h, w2[...]) + b2[...])
    h = _relu(_dot(h, w3[...]) + b3[...])
    h_out[...] = h
    z_out[...] = _dot(h, wm[...])


def _gate_body(h, m3, ld0, ld1, v0, v1, wga, bga, wgb, bgb,
               a0_out, a1_out, y_out, s1_out, s2_out):
    m0 = jnp.clip(m3[0], -20.0, 20.0)
    m1 = jnp.clip(m3[1], -20.0, 20.0)
    hh = h[...]
    zc = jnp.zeros((hh.shape[0], 7), jnp.float32)
    gi0 = jnp.concatenate([hh, m0, ld0[...], zc], axis=1)
    gi1 = jnp.concatenate([hh, m1, ld1[...], zc], axis=1)
    g0 = _relu(_dot(gi0, wga[...]) + bga[...])
    g1 = _relu(_dot(gi1, wga[...]) + bga[...])
    s0 = (_dot(g0, wgb[...]) + bgb[...]) / 0.6
    s1 = (_dot(g1, wgb[...]) + bgb[...]) / 0.6
    mx = jnp.maximum(s0, s1)
    e0 = jnp.exp(s0 - mx)
    e1 = jnp.exp(s1 - mx)
    den = e0 + e1
    a0 = e0 / den * v0[...]
    a1 = e1 / den * v1[...]
    ssum = jnp.maximum(a0 + a1, 1e-12)
    a0 = a0 / ssum
    a1 = a1 / ssum
    a0 = jnp.maximum(a0, 1e-8)
    a1 = jnp.maximum(a1, 1e-8)
    ssum = jnp.maximum(a0 + a1, 1e-12)
    a0 = a0 / ssum
    a1 = a1 / ssum
    y = a0 * m0 + a1 * m1 + hh
    a0_out[...] = a0
    a1_out[...] = a1
    y_out[...] = y

    @pl.when(pl.program_id(0) == 0)
    def _():
        s1_out[...] = jnp.zeros_like(s1_out)
        s2_out[...] = jnp.zeros_like(s2_out)

    s1_out[...] += jnp.sum(y, axis=0, keepdims=True)
    s2_out[...] += jnp.sum(y * y).reshape(1, 1)


def _pairnorm(y_ref, s1_ref, s2_ref):
    mu = s1_ref[...] * (1.0 / _N)
    var = s2_ref[0, 0] * (1.0 / _N) - jnp.sum(mu * mu)
    msn = jnp.sqrt(var) + 1e-6
    return _relu((y_ref[...] - mu) / msn)


def _norm_mid_body(y, s1, s2, wmsg, h1_out, z1_out):
    h1 = _pairnorm(y, s1, s2)
    h1_out[...] = h1
    z1_out[...] = _dot(h1, wmsg[...])


def _norm_post_body(y, s1, s2, wh1, bh1, wh2, bh2, logit_out):
    h2 = _pairnorm(y, s1, s2)
    hh = _relu(_dot(h2, wh1[...]) + bh1[...])
    logit_out[...] = _dot(hh, wh2[...]) + bh2[...]


def _sds(shape):
    return jax.ShapeDtypeStruct(shape, jnp.float32)


def _row_spec(cols):
    return pl.BlockSpec((_BS, cols), lambda i: (i, 0))


def _full_spec(shape):
    nd = len(shape)
    return pl.BlockSpec(shape, lambda i: (0,) * nd)


def _gate_call(h, m3, ld0, ld1, v0, v1, wga, bga, wgb, bgb):
    return pl.pallas_call(
        _gate_body,
        grid=(_GRID,),
        in_specs=[
            _row_spec(_DH),
            pl.BlockSpec((_NSC, _BS, _DH), lambda i: (0, i, 0)),
            _row_spec(1), _row_spec(1), _row_spec(1), _row_spec(1),
            _full_spec((136, _DH)), _full_spec((1, _DH)),
            _full_spec((_DH, 1)), _full_spec((1, 1)),
        ],
        out_specs=[
            _row_spec(1), _row_spec(1), _row_spec(_DH),
            _full_spec((1, _DH)), _full_spec((1, 1)),
        ],
        out_shape=[_sds((_N, 1)), _sds((_N, 1)), _sds((_N, _DH)),
                   _sds((1, _DH)), _sds((1, 1))],
    )(h, m3, ld0, ld1, v0, v1, wga, bga, wgb, bgb)


def kernel(X, edge_index_0, edge_index_1, mask_0, mask_1, logdeg_0, logdeg_1,
           W_in1, b_in1, W_in2, b_in2, W_in3, b_in3, W_msg0, W_msg1,
           Wg0a, bg0a, Wg0b, bg0b, Wg1a, bg1a, Wg1b, bg1b, Wh1, bh1, Wh2, bh2):
    e0 = edge_index_0.astype(jnp.int32).reshape(2, _NROWS, _CHUNK)
    e1 = edge_index_1.astype(jnp.int32).reshape(2, _NROWS, _CHUNK)
    zeros = jnp.zeros((_NPAD, _DH), jnp.float32)

    ld0 = logdeg_0.reshape(_N, 1)
    ld1 = logdeg_1.reshape(_N, 1)
    v0 = mask_0.reshape(_N, 1)
    v1 = mask_1.reshape(_N, 1)
    b1 = b_in1.reshape(1, _DH)
    b2 = b_in2.reshape(1, _DH)
    b3 = b_in3.reshape(1, _DH)
    bg0 = bg0a.reshape(1, _DH)
    bg1 = bg1a.reshape(1, _DH)
    bg0b = bg0b.reshape(1, 1)
    bg1b = bg1b.reshape(1, 1)
    bh1r = bh1.reshape(1, _DH)
    bh2r = bh2.reshape(1, 1)
    wpad = jnp.zeros((7, _DH), jnp.float32)
    wg0p = jnp.concatenate([Wg0a, wpad], axis=0)
    wg1p = jnp.concatenate([Wg1a, wpad], axis=0)

    h, z0 = pl.pallas_call(
        _pre_body,
        grid=(_GRID,),
        in_specs=[
            pl.BlockSpec((_BS, 128), lambda i: (i, 0)),
            _full_spec((128, _DH)), _full_spec((1, _DH)),
            _full_spec((_DH, _DH)), _full_spec((1, _DH)),
            _full_spec((_DH, _DH)), _full_spec((1, _DH)),
            _full_spec((_DH, _DH)),
        ],
        out_specs=[_row_spec(_DH), _row_spec(_DH)],
        out_shape=[_sds((_N, _DH)), _sds((_N, _DH))],
    )(X, W_in1, b1, W_in2, b2, W_in3, b3, W_msg0)

    m0 = _sc_segsum(z0, e0, e1, zeros)

    a00, a01, y0, s1a, s2a = _gate_call(h, m0, ld0, ld1, v0, v1, wg0p, bg0,
                                        Wg0b, bg0b)
    h1, z1 = pl.pallas_call(
        _norm_mid_body,
        grid=(_GRID,),
        in_specs=[
            _row_spec(_DH), _full_spec((1, _DH)), _full_spec((1, 1)),
            _full_spec((_DH, _DH)),
        ],
        out_specs=[_row_spec(_DH), _row_spec(_DH)],
        out_shape=[_sds((_N, _DH)), _sds((_N, _DH))],
    )(y0, s1a, s2a, W_msg1)

    m1 = _sc_segsum(z1, e0, e1, zeros)

    a10, a11, y1, s1b, s2b = _gate_call(h1, m1, ld0, ld1, v0, v1, wg1p, bg1,
                                        Wg1b, bg1b)
    logits, = pl.pallas_call(
        _norm_post_body,
        grid=(_GRID,),
        in_specs=[
            _row_spec(_DH), _full_spec((1, _DH)), _full_spec((1, 1)),
            _full_spec((_DH, _DH)), _full_spec((1, _DH)),
            _full_spec((_DH, 1)), _full_spec((1, 1)),
        ],
        out_specs=[_row_spec(1)],
        out_shape=[_sds((_N, 1))],
    )(y1, s1b, s2b, Wh1, bh1r, Wh2, bh2r)

    alpha0 = jnp.concatenate([a00, a01], axis=1)
    alpha1 = jnp.concatenate([a10, a11], axis=1)
    return logits[:, 0], alpha0, alpha1


# speedup vs baseline: 1.5642x; 1.0293x over previous
import jax
import jax.numpy as jnp
from jax import lax
from jax.experimental import pallas as pl
from jax.experimental.pallas import tpu as pltpu
from jax.experimental.pallas import tpu_sc as plsc

_N = 10000
_E = 320000
_DH = 64
_INV_TEMP = 1.0 / 0.6

_NSC = 2
_NTILE = 16
_CHUNK = 320
_NROWS = _E // _CHUNK
_CBASE = _NROWS // _NTILE
_CEXT = _NROWS % _NTILE
_NCHUNK = _CBASE + 1
_NPAD = _N + 112
_ZROWS = _NPAD // _NTILE


def _relu(x):
    return jnp.maximum(x, 0.0)


def _dot(a, b):
    return jnp.dot(a, b, preferred_element_type=jnp.float32)


def _sc_body(z_hbm, e0_hbm, e1_hbm, zero_hbm, out_hbm, sidx, didx,
             rows, rows_b, msh, sem, sem_b):
    c = lax.axis_index("c")
    s = lax.axis_index("s")
    row0 = s * _CBASE + jnp.minimum(s, _CEXT)
    nchunk = jnp.where(s < _CEXT, _CBASE + 1, _CBASE)

    def stage(e_hbm):
        pltpu.sync_copy(e_hbm.at[0, pl.ds(row0, _CBASE)],
                        sidx.at[pl.ds(0, _CBASE)])
        pltpu.sync_copy(e_hbm.at[1, pl.ds(row0, _CBASE)],
                        didx.at[pl.ds(0, _CBASE)])

        @pl.when(s < _CEXT)
        def _():
            pltpu.sync_copy(e_hbm.at[0, pl.ds(row0 + _CBASE, 1)],
                            sidx.at[pl.ds(_CBASE, 1)])
            pltpu.sync_copy(e_hbm.at[1, pl.ds(row0 + _CBASE, 1)],
                            didx.at[pl.ds(_CBASE, 1)])

    @pl.when(c == 0)
    def _():
        stage(e0_hbm)

    @pl.when(c == 1)
    def _():
        stage(e1_hbm)

    pltpu.sync_copy(zero_hbm.at[pl.ds(s * _ZROWS, _ZROWS)],
                    msh.at[pl.ds(s * _ZROWS, _ZROWS)])
    plsc.subcore_barrier()

    pltpu.async_copy(z_hbm.at[sidx.at[0]], rows, sem)

    def pair(p, carry):
        i0 = 2 * p
        pltpu.make_async_copy(z_hbm.at[sidx.at[i0]], rows, sem).wait()
        pltpu.async_copy(z_hbm.at[sidx.at[i0 + 1]], rows_b, sem_b)
        pltpu.sync_copy(rows, msh.at[didx.at[i0]], add=True)
        pltpu.make_async_copy(z_hbm.at[sidx.at[i0 + 1]], rows_b, sem_b).wait()
        pltpu.async_copy(z_hbm.at[sidx.at[lax.rem(i0 + 2, _CBASE)]], rows, sem)
        pltpu.sync_copy(rows_b, msh.at[didx.at[i0 + 1]], add=True)
        return carry

    lax.fori_loop(0, _CBASE // 2, pair, 0)
    pltpu.make_async_copy(z_hbm.at[sidx.at[0]], rows, sem).wait()

    @pl.when(s < _CEXT)
    def _():
        pltpu.async_copy(z_hbm.at[sidx.at[_CBASE]], rows, sem).wait()
        pltpu.sync_copy(rows, msh.at[didx.at[_CBASE]], add=True)

    plsc.subcore_barrier()

    pltpu.sync_copy(msh.at[pl.ds(s * _ZROWS, _ZROWS)],
                    out_hbm.at[c, pl.ds(s * _ZROWS, _ZROWS)])


def _sc_segsum(z, e0, e1, zeros):
    fn = pl.kernel(
        _sc_body,
        out_type=jax.ShapeDtypeStruct((_NSC, _NPAD, _DH), jnp.float32),
        mesh=plsc.VectorSubcoreMesh(core_axis_name="c", subcore_axis_name="s",
                                    num_cores=_NSC, num_subcores=_NTILE),
        scratch_types=[
            pltpu.VMEM((_NCHUNK, _CHUNK), jnp.int32),
            pltpu.VMEM((_NCHUNK, _CHUNK), jnp.int32),
            pltpu.VMEM((_CHUNK, _DH), jnp.float32),
            pltpu.VMEM((_CHUNK, _DH), jnp.float32),
            pltpu.VMEM_SHARED((_NPAD, _DH), jnp.float32),
            pltpu.SemaphoreType.DMA,
            pltpu.SemaphoreType.DMA,
        ],
        compiler_params=pltpu.CompilerParams(use_tc_tiling_on_sc=False),
    )
    return fn(z, e0, e1, zeros)


_BS = 2000
_GRID = _N // _BS


def _pre_body(x, w1, b1, w2, b2, w3, b3, wm, h_out, z_out):
    h = _relu(_dot(x[...], w1[...]) + b1[...])
    h = _relu(_dot(h, w2[...]) + b2[...])
    h = _relu(_dot(h, w3[...]) + b3[...])
    h_out[...] = h
    z_out[...] = _dot(h, wm[...])


def _gate_body(h, m3, ld0, ld1, v0, v1, wga, bga, wgb, bgb,
               a0_out, a1_out, y_out, s1_out, s2_out):
    m0 = jnp.clip(m3[0], -20.0, 20.0)
    m1 = jnp.clip(m3[1], -20.0, 20.0)
    hh = h[...]
    zc = jnp.zeros((hh.shape[0], 7), jnp.float32)
    gi0 = jnp.concatenate([hh, m0, ld0[...], zc], axis=1)
    gi1 = jnp.concatenate([hh, m1, ld1[...], zc], axis=1)
    g0 = _relu(_dot(gi0, wga[...]) + bga[...])
    g1 = _relu(_dot(gi1, wga[...]) + bga[...])
    s0 = (_dot(g0, wgb[...]) + bgb[...]) / 0.6
    s1 = (_dot(g1, wgb[...]) + bgb[...]) / 0.6
    mx = jnp.maximum(s0, s1)
    e0 = jnp.exp(s0 - mx)
    e1 = jnp.exp(s1 - mx)
    den = e0 + e1
    a0 = e0 / den * v0[...]
    a1 = e1 / den * v1[...]
    ssum = jnp.maximum(a0 + a1, 1e-12)
    a0 = a0 / ssum
    a1 = a1 / ssum
    a0 = jnp.maximum(a0, 1e-8)
    a1 = jnp.maximum(a1, 1e-8)
    ssum = jnp.maximum(a0 + a1, 1e-12)
    a0 = a0 / ssum
    a1 = a1 / ssum
    y = a0 * m0 + a1 * m1 + hh
    a0_out[...] = a0
    a1_out[...] = a1
    y_out[...] = y

    @pl.when(pl.program_id(0) == 0)
    def _():
        s1_out[...] = jnp.zeros_like(s1_out)
        s2_out[...] = jnp.zeros_like(s2_out)

    s1_out[...] += jnp.sum(y, axis=0, keepdims=True)
    s2_out[...] += jnp.sum(y * y).reshape(1, 1)


def _pairnorm(y_ref, s1_ref, s2_ref):
    mu = s1_ref[...] * (1.0 / _N)
    var = s2_ref[0, 0] * (1.0 / _N) - jnp.sum(mu * mu)
    msn = jnp.sqrt(var) + 1e-6
    return _relu((y_ref[...] - mu) / msn)


def _norm_mid_body(y, s1, s2, wmsg, h1_out, z1_out):
    h1 = _pairnorm(y, s1, s2)
    h1_out[...] = h1
    z1_out[...] = _dot(h1, wmsg[...])


def _norm_post_body(y, s1, s2, wh1, bh1, wh2, bh2, logit_out):
    h2 = _pairnorm(y, s1, s2)
    hh = _relu(_dot(h2, wh1[...]) + bh1[...])
    logit_out[...] = _dot(hh, wh2[...]) + bh2[...]


def _sds(shape):
    return jax.ShapeDtypeStruct(shape, jnp.float32)


def _row_spec(cols):
    return pl.BlockSpec((_BS, cols), lambda i: (i, 0))


def _full_spec(shape):
    nd = len(shape)
    return pl.BlockSpec(shape, lambda i: (0,) * nd)


def _gate_call(h, m3, ld0, ld1, v0, v1, wga, bga, wgb, bgb):
    return pl.pallas_call(
        _gate_body,
        grid=(_GRID,),
        in_specs=[
            _row_spec(_DH),
            pl.BlockSpec((_NSC, _BS, _DH), lambda i: (0, i, 0)),
            _row_spec(1), _row_spec(1), _row_spec(1), _row_spec(1),
            _full_spec((136, _DH)), _full_spec((1, _DH)),
            _full_spec((_DH, 1)), _full_spec((1, 1)),
        ],
        out_specs=[
            _row_spec(1), _row_spec(1), _row_spec(_DH),
            _full_spec((1, _DH)), _full_spec((1, 1)),
        ],
        out_shape=[_sds((_N, 1)), _sds((_N, 1)), _sds((_N, _DH)),
                   _sds((1, _DH)), _sds((1, 1))],
    )(h, m3, ld0, ld1, v0, v1, wga, bga, wgb, bgb)


def kernel(X, edge_index_0, edge_index_1, mask_0, mask_1, logdeg_0, logdeg_1,
           W_in1, b_in1, W_in2, b_in2, W_in3, b_in3, W_msg0, W_msg1,
           Wg0a, bg0a, Wg0b, bg0b, Wg1a, bg1a, Wg1b, bg1b, Wh1, bh1, Wh2, bh2):
    e0 = edge_index_0.astype(jnp.int32).reshape(2, _NROWS, _CHUNK)
    e1 = edge_index_1.astype(jnp.int32).reshape(2, _NROWS, _CHUNK)
    zeros = jnp.zeros((_NPAD, _DH), jnp.float32)

    ld0 = logdeg_0.reshape(_N, 1)
    ld1 = logdeg_1.reshape(_N, 1)
    v0 = mask_0.reshape(_N, 1)
    v1 = mask_1.reshape(_N, 1)
    b1 = b_in1.reshape(1, _DH)
    b2 = b_in2.reshape(1, _DH)
    b3 = b_in3.reshape(1, _DH)
    bg0 = bg0a.reshape(1, _DH)
    bg1 = bg1a.reshape(1, _DH)
    bg0b = bg0b.reshape(1, 1)
    bg1b = bg1b.reshape(1, 1)
    bh1r = bh1.reshape(1, _DH)
    bh2r = bh2.reshape(1, 1)
    wpad = jnp.zeros((7, _DH), jnp.float32)
    wg0p = jnp.concatenate([Wg0a, wpad], axis=0)
    wg1p = jnp.concatenate([Wg1a, wpad], axis=0)

    h, z0 = pl.pallas_call(
        _pre_body,
        grid=(_GRID,),
        in_specs=[
            pl.BlockSpec((_BS, 128), lambda i: (i, 0)),
            _full_spec((128, _DH)), _full_spec((1, _DH)),
            _full_spec((_DH, _DH)), _full_spec((1, _DH)),
            _full_spec((_DH, _DH)), _full_spec((1, _DH)),
            _full_spec((_DH, _DH)),
        ],
        out_specs=[_row_spec(_DH), _row_spec(_DH)],
        out_shape=[_sds((_N, _DH)), _sds((_N, _DH))],
    )(X, W_in1, b1, W_in2, b2, W_in3, b3, W_msg0)

    m0 = _sc_segsum(z0, e0, e1, zeros)

    a00, a01, y0, s1a, s2a = _gate_call(h, m0, ld0, ld1, v0, v1, wg0p, bg0,
                                        Wg0b, bg0b)
    h1, z1 = pl.pallas_call(
        _norm_mid_body,
        grid=(_GRID,),
        in_specs=[
            _row_spec(_DH), _full_spec((1, _DH)), _full_spec((1, 1)),
            _full_spec((_DH, _DH)),
        ],
        out_specs=[_row_spec(_DH), _row_spec(_DH)],
        out_shape=[_sds((_N, _DH)), _sds((_N, _DH))],
    )(y0, s1a, s2a, W_msg1)

    m1 = _sc_segsum(z1, e0, e1, zeros)

    a10, a11, y1, s1b, s2b = _gate_call(h1, m1, ld0, ld1, v0, v1, wg1p, bg1,
                                        Wg1b, bg1b)
    logits, = pl.pallas_call(
        _norm_post_body,
        grid=(_GRID,),
        in_specs=[
            _row_spec(_DH), _full_spec((1, _DH)), _full_spec((1, 1)),
            _full_spec((_DH, _DH)), _full_spec((1, _DH)),
            _full_spec((_DH, 1)), _full_spec((1, 1)),
        ],
        out_specs=[_row_spec(1)],
        out_shape=[_sds((_N, 1))],
    )(y1, s1b, s2b, Wh1, bh1r, Wh2, bh2r)

    alpha0 = jnp.concatenate([a00, a01], axis=1)
    alpha1 = jnp.concatenate([a10, a11], axis=1)
    return logits[:, 0], alpha0, alpha1
